# baseline (speedup 1.0000x reference)
import functools

import jax
import jax.numpy as jnp
from jax import lax
from jax.experimental import pallas as pl
from jax.experimental.pallas import tpu as pltpu
from jax.experimental.pallas import tpu_sc as plsc

_NC = 2
_NS = 16
_NW = _NC * _NS
_CHUNK = 128
_G = 8
_L = 16


@functools.cache
def _build(B, S, D, V):
    N = B * S
    T = N // _NW
    n_groups = D // _G
    n_chunks = T // _CHUNK
    mesh = plsc.VectorSubcoreMesh(
        core_axis_name="c", subcore_axis_name="s",
        num_cores=_NC, num_subcores=_NS)

    @functools.partial(
        pl.kernel,
        out_type=jax.ShapeDtypeStruct((B, D, S), jnp.float32),
        mesh=mesh,
        scratch_types=[
            pltpu.VMEM((_G, T), jnp.int32),
            pltpu.VMEM((_G, T), jnp.float32),
            pltpu.VMEM((_G, T), jnp.float32),
            pltpu.SemaphoreType.DMA,
        ],
    )
    def _k(idxs_hbm, tok_hbm, posT_hbm, out_hbm, idx_v, acc_v, pos_v, sem):
        wid = lax.axis_index("s") * _NC + lax.axis_index("c")
        tok0 = wid * T
        b = lax.div(tok0, S)
        s0 = lax.rem(tok0, S)

        def group_body(g, carry):
            g8 = g * _G
            pltpu.sync_copy(idxs_hbm.at[wid, pl.ds(g8, _G)], idx_v)
            pltpu.sync_copy(posT_hbm.at[pl.ds(g8, _G), pl.ds(s0, T)], pos_v)
            copies = []
            for dd in range(_G):
                for c in range(n_chunks):
                    copies.append(pltpu.async_copy(
                        tok_hbm.at[idx_v.at[dd, pl.ds(c * _CHUNK, _CHUNK)]],
                        acc_v.at[dd, pl.ds(c * _CHUNK, _CHUNK)],
                        sem))
            for cp in copies:
                cp.wait()
            for dd in range(_G):
                for c in range(n_chunks):
                    sl = pl.ds(c * _CHUNK, _CHUNK)
                    acc_v[dd, sl] = acc_v[dd, sl] + pos_v[dd, sl]
            pltpu.sync_copy(
                acc_v, out_hbm.at[b, pl.ds(g8, _G), pl.ds(s0, T)])
            return carry

        lax.fori_loop(0, n_groups, group_body, 0)

    return _k


def kernel(input_ids, token_table, pos_table):
    B, S = input_ids.shape
    V, D = token_table.shape
    N = B * S
    T = N // _NW
    tok_flat = token_table.T.reshape(V * D)
    posT = pos_table.T
    ids = input_ids.reshape(_NW, 1, T).astype(jnp.int32)
    idxs = ids + (jnp.arange(D, dtype=jnp.int32) * V)[None, :, None]
    out = _build(B, S, D, V)(idxs, tok_flat, posT)
    return out.transpose(0, 2, 1)


# speedup vs baseline: 7.5833x; 7.5833x over previous
import functools

import jax
import jax.numpy as jnp
from jax import lax
from jax.experimental import pallas as pl
from jax.experimental.pallas import tpu as pltpu
from jax.experimental.pallas import tpu_sc as plsc

_NC = 2
_NS = 16
_NW = _NC * _NS
_C = 128
_L = 16


@functools.cache
def _build(B, S, D, V):
    N = B * S
    T = N // _NW
    n_chunks = T // _C
    R = 2 * D
    mesh = plsc.VectorSubcoreMesh(
        core_axis_name="c", subcore_axis_name="s",
        num_cores=_NC, num_subcores=_NS)

    @functools.partial(
        pl.kernel,
        out_type=jax.ShapeDtypeStruct((N // _C, _C // 2, R), jnp.float32),
        mesh=mesh,
        scratch_types=[
            pltpu.VMEM((n_chunks, _C), jnp.int32),
            pltpu.VMEM((2, _C, R), jnp.float32),
            pltpu.VMEM((2, _C, _L), jnp.float32),
            pltpu.VMEM((2, _C // 2, R), jnp.float32),
            pltpu.VMEM((2, _C // 2, R), jnp.float32),
            pltpu.SemaphoreType.DMA,
            pltpu.SemaphoreType.DMA,
            pltpu.SemaphoreType.DMA,
            pltpu.SemaphoreType.DMA,
            pltpu.SemaphoreType.DMA,
            pltpu.SemaphoreType.DMA,
            pltpu.SemaphoreType.DMA,
            pltpu.SemaphoreType.DMA,
        ],
    )
    def _k(rows_hbm, par_hbm, tok_hbm, pos_hbm, out_hbm,
           rows_v, buf_v, par_v, pos_v, acc_v,
           g0, g1, p0, p1, q0, q1, o0, o1):
        gsem = (g0, g1)
        psem = (p0, p1)
        qsem = (q0, q1)
        osem = (o0, o1)
        wid = lax.axis_index("s") * _NC + lax.axis_index("c")
        pos0 = lax.mul(lax.rem(wid, S // T), n_chunks)
        out0 = lax.mul(wid, n_chunks)
        pltpu.sync_copy(rows_hbm.at[wid], rows_v)

        def issue(c):
            s = c & 1
            g = pltpu.async_copy(
                tok_hbm.at[rows_v.at[c]], buf_v.at[s], gsem[s])
            p = pltpu.async_copy(
                pos_hbm.at[pos0 + c], pos_v.at[s], psem[s])
            q = pltpu.async_copy(
                par_hbm.at[wid, c], par_v.at[s], qsem[s])
            return g, p, q

        def extract(c):
            s = c & 1

            def tok_body(j, carry):
                pm = par_v[s, j]
                r = lax.shift_right_logical(j, 1)
                h = lax.mul(lax.bitwise_and(j, 1), D)
                for grp in range(D // _L):
                    lo = buf_v[s, j, pl.ds(grp * _L, _L)]
                    hi = buf_v[s, j, pl.ds(D + grp * _L, _L)]
                    val = lo * (1.0 - pm) + hi * pm
                    sl = pl.ds(h + grp * _L, _L)
                    acc_v[s, r, sl] = val + pos_v[s, r, sl]
                return carry

            lax.fori_loop(0, _C, tok_body, 0)

        pending = {0: issue(0), 1: issue(1)}
        out_pending = {}
        for c in range(n_chunks):
            g, p, q = pending.pop(c)
            g.wait()
            p.wait()
            q.wait()
            if c >= 2:
                out_pending.pop(c - 2).wait()
            extract(c)
            out_pending[c] = pltpu.async_copy(
                acc_v.at[c & 1], out_hbm.at[out0 + c], osem[c & 1])
            if c + 2 < n_chunks:
                pending[c + 2] = issue(c + 2)
        out_pending.pop(n_chunks - 2).wait()
        out_pending.pop(n_chunks - 1).wait()

    return _k


def kernel(input_ids, token_table, pos_table):
    B, S = input_ids.shape
    V, D = token_table.shape
    N = B * S
    T = N // _NW
    n_chunks = T // _C
    ids = input_ids.reshape(_NW, n_chunks, _C).astype(jnp.int32)
    rows = lax.shift_right_logical(ids, 1)
    par = jnp.broadcast_to(
        jnp.bitwise_and(ids, 1).astype(jnp.float32)[..., None],
        (_NW, n_chunks, _C, _L))
    tok2 = token_table.reshape(V // 2, 2 * D)
    pos3 = pos_table.reshape(S // _C, _C // 2, 2 * D)
    out3 = _build(B, S, D, V)(rows, par, tok2, pos3)
    return out3.reshape(B, S, D)
